# baseline (speedup 1.0000x reference)
import functools

import jax
import jax.numpy as jnp
from jax.experimental import pallas as pl
from jax.experimental.pallas import tpu as pltpu

WINDOW = 128
HALF = WINDOW // 2
PIX = WINDOW * WINDOW


def _mlp_kernel(tops_ref, lefts_ref, rowb_ref, colb_ref, x_ref, sig_ref,
                w1t_ref, b1c_ref, w2t_ref, b2_ref, w3_ref, b3_ref,
                out_ref, iidd_ref, xs8):
    i = pl.program_id(0)

    @pl.when(i == 0)
    def _():
        xs_bf = jnp.concatenate([x_ref[...], sig_ref[...]],
                                axis=0).astype(jnp.bfloat16)
        for sh in range(16):
            xs8[sh] = pltpu.roll(xs_bf, (512 - sh) % 512, 1)

    top = tops_ref[i]
    left = lefts_ref[i]
    base = pl.multiple_of((top // 16) * 16, 16)
    s = top - base

    slab = xs8[s, :, pl.ds(base, WINDOW), :]
    crop = pltpu.roll(slab, (512 - left) % 512, 2)[:, :, :WINDOW]
    feat = crop.reshape(4, PIX)

    h1 = jnp.dot(w1t_ref[...], feat, preferred_element_type=jnp.float32)
    h1 = jnp.maximum(h1.astype(jnp.bfloat16) + b1c_ref[0, 0, :][:, None],
                     jnp.bfloat16(0))
    h2 = jnp.dot(w2t_ref[...], h1, preferred_element_type=jnp.float32)
    h2 = jnp.maximum(h2.astype(jnp.bfloat16) + b2_ref[...], jnp.bfloat16(0))
    o = jnp.dot(w3_ref[...], h2, preferred_element_type=jnp.float32)
    prob = 1.0 / (1.0 + jnp.exp(-(o + b3_ref[0, 0])))
    out_ref[...] = prob.reshape(WINDOW, WINDOW)

    iidd_ref[0, :] = jnp.full((PIX,), i, jnp.int32)
    iidd_ref[1, :] = rowb_ref[0, :] + top
    iidd_ref[2, :] = colb_ref[0, :] + left


@jax.jit
def kernel(x, sigma, c, centroids_idx, W1, b1, W2, b2, W3, b3):
    H, W = x.shape[-2:]
    C = c.shape[0]
    E = x.shape[0]

    tops = jnp.clip(centroids_idx[:, 0], HALF, H - HALF) - HALF
    lefts = jnp.clip(centroids_idx[:, 1], HALF, W - HALF) - HALF

    w1t = W1.T.astype(jnp.bfloat16)
    w2t = W2.T.astype(jnp.bfloat16)
    w3row = W3.T.astype(jnp.bfloat16)
    b1c = (b1[None, :] - c @ W1[:E]).astype(jnp.bfloat16).reshape(C, 1, 64)
    b2col = b2.astype(jnp.bfloat16).reshape(-1, 1)
    b3r = b3.reshape(1, 1)
    p = jnp.arange(PIX, dtype=jnp.int32)
    rowb = (p // WINDOW).reshape(1, PIX)
    colb = (p % WINDOW).reshape(1, PIX)

    out_shape = (
        jax.ShapeDtypeStruct((C * WINDOW, WINDOW), jnp.float32),
        jax.ShapeDtypeStruct((3, C * PIX), jnp.int32),
    )
    smem = functools.partial(pl.BlockSpec, memory_space=pltpu.SMEM)
    vmem_full = lambda shp: pl.BlockSpec(shp, lambda i: (0,) * len(shp))
    prob, iidd4 = pl.pallas_call(
        _mlp_kernel,
        grid=(C,),
        in_specs=[
            smem((C,), lambda i: (0,)),
            smem((C,), lambda i: (0,)),
            vmem_full((1, PIX)),
            vmem_full((1, PIX)),
            vmem_full((E, H, W)),
            vmem_full((E, H, W)),
            vmem_full((64, 4)),
            pl.BlockSpec((1, 1, 64), lambda i: (i, 0, 0)),
            vmem_full((64, 64)),
            vmem_full((64, 1)),
            vmem_full((1, 64)),
            vmem_full((1, 1)),
        ],
        out_specs=(
            pl.BlockSpec((WINDOW, WINDOW), lambda i: (i, 0)),
            pl.BlockSpec((3, PIX), lambda i: (0, i)),
        ),
        out_shape=out_shape,
        scratch_shapes=[
            pltpu.VMEM((16, 4, H, W), jnp.bfloat16),
        ],
    )(tops, lefts, rowb, colb, x, sigma, w1t, b1c, w2t, b2col, w3row, b3r)

    prob = prob.reshape(C, 1, WINDOW, WINDOW)
    return (prob, iidd4)


# speedup vs baseline: 1.1072x; 1.1072x over previous
import functools

import jax
import jax.numpy as jnp
from jax.experimental import pallas as pl
from jax.experimental.pallas import tpu as pltpu

WINDOW = 128
HALF = WINDOW // 2
PIX = WINDOW * WINDOW
PERPROG = 2


def _mlp_kernel(tops_ref, lefts_ref, rowb_ref, colb_ref, x_ref, sig_ref,
                w1t_ref, b1c_ref, w2t_ref, b2_ref, w3_ref, b3_ref,
                out_ref, iidd_ref, xs8):
    i = pl.program_id(0)

    @pl.when(i == 0)
    def _():
        xs_bf = jnp.concatenate([x_ref[...], sig_ref[...]],
                                axis=0).astype(jnp.bfloat16)
        for sh in range(16):
            xs8[sh] = pltpu.roll(xs_bf, (512 - sh) % 512, 1)

    for j in range(PERPROG):
        ci = i * PERPROG + j
        top = tops_ref[ci]
        left = lefts_ref[ci]
        base = pl.multiple_of((top // 16) * 16, 16)
        s = top - base

        slab = xs8[s, :, pl.ds(base, WINDOW), :]
        crop = pltpu.roll(slab, (512 - left) % 512, 2)[:, :, :WINDOW]
        feat = crop.reshape(4, PIX)

        h1 = jnp.dot(w1t_ref[...], feat, preferred_element_type=jnp.float32)
        h1 = jnp.maximum(h1 + b1c_ref[j, 0, :][:, None], 0).astype(jnp.bfloat16)
        h2 = jnp.dot(w2t_ref[...], h1, preferred_element_type=jnp.float32)
        h2 = jnp.maximum(h2 + b2_ref[...], 0).astype(jnp.bfloat16)
        o = jnp.dot(w3_ref[...], h2, preferred_element_type=jnp.float32)
        prob = 1.0 / (1.0 + jnp.exp(-(o + b3_ref[0, 0])))
        out_ref[j * WINDOW:(j + 1) * WINDOW, :] = prob.reshape(WINDOW, WINDOW)

        iidd_ref[0, j * PIX:(j + 1) * PIX] = jnp.full((PIX,), ci, jnp.int32)
        iidd_ref[1, j * PIX:(j + 1) * PIX] = rowb_ref[0, :] + top
        iidd_ref[2, j * PIX:(j + 1) * PIX] = colb_ref[0, :] + left


@jax.jit
def kernel(x, sigma, c, centroids_idx, W1, b1, W2, b2, W3, b3):
    H, W = x.shape[-2:]
    C = c.shape[0]
    E = x.shape[0]

    tops = jnp.clip(centroids_idx[:, 0], HALF, H - HALF) - HALF
    lefts = jnp.clip(centroids_idx[:, 1], HALF, W - HALF) - HALF

    w1t = W1.T.astype(jnp.bfloat16)
    w2t = W2.T.astype(jnp.bfloat16)
    w3row = W3.T.astype(jnp.bfloat16)
    b1c = (b1[None, :] - c @ W1[:E]).astype(jnp.bfloat16).reshape(C, 1, 64)
    b2col = b2.astype(jnp.bfloat16).reshape(-1, 1)
    b3r = b3.reshape(1, 1)
    p = jnp.arange(PIX, dtype=jnp.int32)
    rowb = (p // WINDOW).reshape(1, PIX)
    colb = (p % WINDOW).reshape(1, PIX)

    out_shape = (
        jax.ShapeDtypeStruct((C * WINDOW, WINDOW), jnp.float32),
        jax.ShapeDtypeStruct((3, C * PIX), jnp.int32),
    )
    smem = functools.partial(pl.BlockSpec, memory_space=pltpu.SMEM)
    vmem_full = lambda shp: pl.BlockSpec(shp, lambda i: (0,) * len(shp))
    prob, iidd4 = pl.pallas_call(
        _mlp_kernel,
        grid=(C // PERPROG,),
        in_specs=[
            smem((C,), lambda i: (0,)),
            smem((C,), lambda i: (0,)),
            vmem_full((1, PIX)),
            vmem_full((1, PIX)),
            vmem_full((E, H, W)),
            vmem_full((E, H, W)),
            vmem_full((64, 4)),
            pl.BlockSpec((PERPROG, 1, 64), lambda i: (i, 0, 0)),
            vmem_full((64, 64)),
            vmem_full((64, 1)),
            vmem_full((1, 64)),
            vmem_full((1, 1)),
        ],
        out_specs=(
            pl.BlockSpec((PERPROG * WINDOW, WINDOW), lambda i: (i, 0)),
            pl.BlockSpec((3, PERPROG * PIX), lambda i: (0, i)),
        ),
        out_shape=out_shape,
        scratch_shapes=[
            pltpu.VMEM((16, 4, H, W), jnp.bfloat16),
        ],
    )(tops, lefts, rowb, colb, x, sigma, w1t, b1c, w2t, b2col, w3row, b3r)

    prob = prob.reshape(C, 1, WINDOW, WINDOW)
    return (prob, iidd4)


# speedup vs baseline: 1.1597x; 1.0474x over previous
import functools

import jax
import jax.numpy as jnp
from jax.experimental import pallas as pl
from jax.experimental.pallas import tpu as pltpu

WINDOW = 128
HALF = WINDOW // 2
PIX = WINDOW * WINDOW
PERPROG = 4


def _mlp_kernel(tops_ref, lefts_ref, rowb_ref, colb_ref, x_ref, sig_ref,
                w1t_ref, b1c_ref, w2t_ref, b2_ref, w3_ref, b3_ref,
                out_ref, iidd_ref, xs8):
    i = pl.program_id(0)

    @pl.when(i == 0)
    def _():
        xs_bf = jnp.concatenate([x_ref[...], sig_ref[...]],
                                axis=0).astype(jnp.bfloat16)
        for sh in range(16):
            xs8[sh] = pltpu.roll(xs_bf, (512 - sh) % 512, 1)

    for j in range(PERPROG):
        ci = i * PERPROG + j
        top = tops_ref[ci]
        left = lefts_ref[ci]
        base = pl.multiple_of((top // 16) * 16, 16)
        s = top - base

        slab = xs8[s, :, pl.ds(base, WINDOW), :]
        crop = pltpu.roll(slab, (512 - left) % 512, 2)[:, :, :WINDOW]
        feat = crop.reshape(4, PIX)

        h1 = jnp.dot(w1t_ref[...], feat, preferred_element_type=jnp.float32)
        h1 = jnp.maximum(h1 + b1c_ref[j, 0, :][:, None], 0).astype(jnp.bfloat16)
        h2 = jnp.dot(w2t_ref[...], h1, preferred_element_type=jnp.float32)
        h2 = jnp.maximum(h2 + b2_ref[...], 0).astype(jnp.bfloat16)
        o = jnp.dot(w3_ref[...], h2, preferred_element_type=jnp.float32)
        prob = 1.0 / (1.0 + jnp.exp(-(o + b3_ref[0, 0])))
        out_ref[j * WINDOW:(j + 1) * WINDOW, :] = prob.reshape(WINDOW, WINDOW)

        iidd_ref[0, j * PIX:(j + 1) * PIX] = jnp.full((PIX,), ci, jnp.int32)
        iidd_ref[1, j * PIX:(j + 1) * PIX] = rowb_ref[0, :] + top
        iidd_ref[2, j * PIX:(j + 1) * PIX] = colb_ref[0, :] + left


@jax.jit
def kernel(x, sigma, c, centroids_idx, W1, b1, W2, b2, W3, b3):
    H, W = x.shape[-2:]
    C = c.shape[0]
    E = x.shape[0]

    tops = jnp.clip(centroids_idx[:, 0], HALF, H - HALF) - HALF
    lefts = jnp.clip(centroids_idx[:, 1], HALF, W - HALF) - HALF

    w1t = W1.T.astype(jnp.bfloat16)
    w2t = W2.T.astype(jnp.bfloat16)
    w3row = W3.T.astype(jnp.bfloat16)
    b1c = (b1[None, :] - c @ W1[:E]).astype(jnp.bfloat16).reshape(C, 1, 64)
    b2col = b2.astype(jnp.bfloat16).reshape(-1, 1)
    b3r = b3.reshape(1, 1)
    p = jnp.arange(PIX, dtype=jnp.int32)
    rowb = (p // WINDOW).reshape(1, PIX)
    colb = (p % WINDOW).reshape(1, PIX)

    out_shape = (
        jax.ShapeDtypeStruct((C * WINDOW, WINDOW), jnp.float32),
        jax.ShapeDtypeStruct((3, C * PIX), jnp.int32),
    )
    smem = functools.partial(pl.BlockSpec, memory_space=pltpu.SMEM)
    vmem_full = lambda shp: pl.BlockSpec(shp, lambda i: (0,) * len(shp))
    prob, iidd4 = pl.pallas_call(
        _mlp_kernel,
        grid=(C // PERPROG,),
        in_specs=[
            smem((C,), lambda i: (0,)),
            smem((C,), lambda i: (0,)),
            vmem_full((1, PIX)),
            vmem_full((1, PIX)),
            vmem_full((E, H, W)),
            vmem_full((E, H, W)),
            vmem_full((64, 4)),
            pl.BlockSpec((PERPROG, 1, 64), lambda i: (i, 0, 0)),
            vmem_full((64, 64)),
            vmem_full((64, 1)),
            vmem_full((1, 64)),
            vmem_full((1, 1)),
        ],
        out_specs=(
            pl.BlockSpec((PERPROG * WINDOW, WINDOW), lambda i: (i, 0)),
            pl.BlockSpec((3, PERPROG * PIX), lambda i: (0, i)),
        ),
        out_shape=out_shape,
        scratch_shapes=[
            pltpu.VMEM((16, 4, H, W), jnp.bfloat16),
        ],
    )(tops, lefts, rowb, colb, x, sigma, w1t, b1c, w2t, b2col, w3row, b3r)

    prob = prob.reshape(C, 1, WINDOW, WINDOW)
    return (prob, iidd4)


# speedup vs baseline: 1.1696x; 1.0085x over previous
import functools

import jax
import jax.numpy as jnp
from jax.experimental import pallas as pl
from jax.experimental.pallas import tpu as pltpu

WINDOW = 128
HALF = WINDOW // 2
PIX = WINDOW * WINDOW
PERPROG = 8


def _mlp_kernel(tops_ref, lefts_ref, rowb_ref, colb_ref, x_ref, sig_ref,
                w1t_ref, b1c_ref, w2t_ref, b2_ref, w3_ref, b3_ref,
                out_ref, iidd_ref, xs8):
    i = pl.program_id(0)

    @pl.when(i == 0)
    def _():
        xs_bf = jnp.concatenate([x_ref[...], sig_ref[...]],
                                axis=0).astype(jnp.bfloat16)
        for sh in range(16):
            xs8[sh] = pltpu.roll(xs_bf, (512 - sh) % 512, 1)

    for j in range(PERPROG):
        ci = i * PERPROG + j
        top = tops_ref[ci]
        left = lefts_ref[ci]
        base = pl.multiple_of((top // 16) * 16, 16)
        s = top - base

        slab = xs8[s, :, pl.ds(base, WINDOW), :]
        crop = pltpu.roll(slab, (512 - left) % 512, 2)[:, :, :WINDOW]
        feat = crop.reshape(4, PIX)

        h1 = jnp.dot(w1t_ref[...], feat, preferred_element_type=jnp.float32)
        h1 = jnp.maximum(h1 + b1c_ref[j, 0, :][:, None], 0).astype(jnp.bfloat16)
        h2 = jnp.dot(w2t_ref[...], h1, preferred_element_type=jnp.float32)
        h2 = jnp.maximum(h2 + b2_ref[...], 0).astype(jnp.bfloat16)
        o = jnp.dot(w3_ref[...], h2, preferred_element_type=jnp.float32)
        prob = 1.0 / (1.0 + jnp.exp(-(o + b3_ref[0, 0])))
        out_ref[j * WINDOW:(j + 1) * WINDOW, :] = prob.reshape(WINDOW, WINDOW)

        iidd_ref[0, j * PIX:(j + 1) * PIX] = jnp.full((PIX,), ci, jnp.int32)
        iidd_ref[1, j * PIX:(j + 1) * PIX] = rowb_ref[0, :] + top
        iidd_ref[2, j * PIX:(j + 1) * PIX] = colb_ref[0, :] + left


@jax.jit
def kernel(x, sigma, c, centroids_idx, W1, b1, W2, b2, W3, b3):
    H, W = x.shape[-2:]
    C = c.shape[0]
    E = x.shape[0]

    tops = jnp.clip(centroids_idx[:, 0], HALF, H - HALF) - HALF
    lefts = jnp.clip(centroids_idx[:, 1], HALF, W - HALF) - HALF

    w1t = W1.T.astype(jnp.bfloat16)
    w2t = W2.T.astype(jnp.bfloat16)
    w3row = W3.T.astype(jnp.bfloat16)
    b1c = (b1[None, :] - c @ W1[:E]).astype(jnp.bfloat16).reshape(C, 1, 64)
    b2col = b2.astype(jnp.bfloat16).reshape(-1, 1)
    b3r = b3.reshape(1, 1)
    p = jnp.arange(PIX, dtype=jnp.int32)
    rowb = (p // WINDOW).reshape(1, PIX)
    colb = (p % WINDOW).reshape(1, PIX)

    out_shape = (
        jax.ShapeDtypeStruct((C * WINDOW, WINDOW), jnp.float32),
        jax.ShapeDtypeStruct((3, C * PIX), jnp.int32),
    )
    smem = functools.partial(pl.BlockSpec, memory_space=pltpu.SMEM)
    vmem_full = lambda shp: pl.BlockSpec(shp, lambda i: (0,) * len(shp))
    prob, iidd4 = pl.pallas_call(
        _mlp_kernel,
        grid=(C // PERPROG,),
        in_specs=[
            smem((C,), lambda i: (0,)),
            smem((C,), lambda i: (0,)),
            vmem_full((1, PIX)),
            vmem_full((1, PIX)),
            vmem_full((E, H, W)),
            vmem_full((E, H, W)),
            vmem_full((64, 4)),
            pl.BlockSpec((PERPROG, 1, 64), lambda i: (i, 0, 0)),
            vmem_full((64, 64)),
            vmem_full((64, 1)),
            vmem_full((1, 64)),
            vmem_full((1, 1)),
        ],
        out_specs=(
            pl.BlockSpec((PERPROG * WINDOW, WINDOW), lambda i: (i, 0)),
            pl.BlockSpec((3, PERPROG * PIX), lambda i: (0, i)),
        ),
        out_shape=out_shape,
        scratch_shapes=[
            pltpu.VMEM((16, 4, H, W), jnp.bfloat16),
        ],
    )(tops, lefts, rowb, colb, x, sigma, w1t, b1c, w2t, b2col, w3row, b3r)

    prob = prob.reshape(C, 1, WINDOW, WINDOW)
    return (prob, iidd4)
